# Initial kernel scaffold; baseline (speedup 1.0000x reference)
import jax
import jax.numpy as jnp
from jax.experimental import pallas as pl
from jax.experimental.pallas import tpu as pltpu

_N = 2048
_H = 16
_BR = 128
_NT = _N // _BR


def _tc_body(bt_ref, out_ref, d_scr, s_scr):
    i_blk = pl.program_id(1)

    @pl.when(i_blk == 0)
    def _build():
        d_scr[0:1, 0:1047] = jnp.broadcast_to(bt_ref[0:1, 0:1], (1, 1047))
        d_scr[0:1, 1047:3047] = bt_ref[0:1, 0:2000]
        d_scr[0:1, 3047:4096] = jnp.broadcast_to(bt_ref[0:1, 1999:2000], (1, 1049))
        for r in range(_BR):
            s_scr[r : r + 1, 0 : 2 * _N - _BR] = d_scr[0:1, pl.ds(_BR - 1 - r, 2 * _N - _BR)]

    off = _BR * (_NT - 1 - i_blk)
    out_ref[0, 0, :, :] = s_scr[:, pl.ds(off, _N)]


def kernel(x, bias_values):
    del x
    bt = jnp.transpose(bias_values)
    bt = jnp.pad(bt, ((0, 0), (0, 48)))

    out = pl.pallas_call(
        _tc_body,
        grid=(_H, _NT),
        in_specs=[pl.BlockSpec((1, _N), lambda h, i: (h, 0))],
        out_specs=pl.BlockSpec((1, 1, _BR, _N), lambda h, i: (0, h, i, 0)),
        out_shape=jax.ShapeDtypeStruct((1, _H, _N, _N), jnp.float32),
        scratch_shapes=[
            pltpu.VMEM((1, 2 * _N), jnp.float32),
            pltpu.VMEM((_BR, 2 * _N), jnp.float32),
        ],
    )(bt)
    return out


# speedup vs baseline: 105.7185x; 105.7185x over previous
import jax
import jax.numpy as jnp
from jax.experimental import pallas as pl
from jax.experimental.pallas import tpu as pltpu

_N = 2048
_H = 16
_BR = 128
_NT = _N // _BR


def _tc_body(bt_ref, out_ref, d_scr, s_scr):
    i_blk = pl.program_id(1)

    @pl.when(i_blk == 0)
    def _build():
        d_scr[0:1, 0:1047] = jnp.broadcast_to(bt_ref[0, 0:1, 0:1], (1, 1047))
        d_scr[0:1, 1047:3047] = bt_ref[0, 0:1, 0:2000]
        d_scr[0:1, 3047:4096] = jnp.broadcast_to(bt_ref[0, 0:1, 1999:2000], (1, 1049))
        for r in range(_BR):
            s_scr[r : r + 1, 0 : 2 * _N - _BR] = d_scr[0:1, pl.ds(_BR - 1 - r, 2 * _N - _BR)]

    off = _BR * (_NT - 1 - i_blk)
    out_ref[0, 0, :, :] = s_scr[:, pl.ds(off, _N)]


def kernel(x, bias_values):
    del x
    bt = jnp.transpose(bias_values)
    bt = jnp.pad(bt, ((0, 0), (0, 48)))
    bt = bt.reshape(_H, 1, _N)

    out = pl.pallas_call(
        _tc_body,
        grid=(_H, _NT),
        in_specs=[pl.BlockSpec((1, 1, _N), lambda h, i: (h, 0, 0))],
        out_specs=pl.BlockSpec((1, 1, _BR, _N), lambda h, i: (0, h, i, 0)),
        out_shape=jax.ShapeDtypeStruct((1, _H, _N, _N), jnp.float32),
        scratch_shapes=[
            pltpu.VMEM((1, 2 * _N), jnp.float32),
            pltpu.VMEM((_BR, 2 * _N), jnp.float32),
        ],
    )(bt)
    return out
